# Initial kernel scaffold; baseline (speedup 1.0000x reference)
import jax
import jax.numpy as jnp
from jax.experimental import pallas as pl


def kernel(x, weight, bias):
    raise NotImplementedError("write your pallas kernel here")


# speedup vs baseline: 8.6987x; 8.6987x over previous
import functools

import jax
import jax.numpy as jnp
from jax import lax
from jax.experimental import pallas as pl

N = 32768
DIM = 768
E = 64
TOPK = 8
G = 8
EPG = E // G
TOPK_G = 4
ROUTE_SCALE = 2.5

T = 512

NEG = float("-inf")


def _gate_kernel(x_ref, w_ref, b_ref, wout_ref, iout_ref):
    x = x_ref[...]
    w = w_ref[...]
    b = b_ref[...]

    logits = lax.dot_general(w, x, (((1,), (1,)), ((), ())),
                             preferred_element_type=jnp.float32)
    sig = jax.nn.sigmoid(logits)
    sel = sig + b

    sig3 = sig.reshape(G, EPG, T)
    sel3 = sel.reshape(G, EPG, T)

    ir = lax.broadcasted_iota(jnp.int32, (G, EPG, T), 1)

    m1 = jnp.max(sel3, axis=1, keepdims=True)
    i1 = jnp.min(jnp.where(sel3 == m1, ir, EPG), axis=1, keepdims=True)
    m2 = jnp.max(jnp.where(ir == i1, NEG, sel3), axis=1, keepdims=True)
    gs = (m1 + m2).reshape(G, T)

    rank = jnp.zeros((G, T), dtype=jnp.int32)
    for h in range(G):
        gh = gs[h:h + 1, :]
        beats = (gh > gs).astype(jnp.int32)
        ties = jnp.where(gs == gh, 1, 0)
        tie_lt = jnp.concatenate(
            [jnp.zeros((h + 1, T), jnp.int32), ties[h + 1:, :]], axis=0
        ) if h + 1 < G else jnp.zeros((G, T), jnp.int32)
        rank = rank + beats + tie_lt
    keep = rank < TOPK_G

    sm = jnp.where(keep[:, None, :], sel3, NEG)

    ig = lax.broadcasted_iota(jnp.int32, (G, EPG, T), 0)
    eid = ig * EPG + ir

    vals = []
    idxs = []
    for _ in range(TOPK):
        m = jnp.max(jnp.max(sm, axis=1, keepdims=True), axis=0, keepdims=True)
        cand = jnp.where(sm == m, eid, E)
        ij = jnp.min(jnp.min(cand, axis=1, keepdims=True), axis=0,
                     keepdims=True)
        hit = eid == ij
        vo = jnp.max(jnp.max(jnp.where(hit, sig3, NEG), axis=1, keepdims=True),
                     axis=0, keepdims=True)
        sm = jnp.where(hit, NEG, sm)
        vals.append(vo.reshape(1, T))
        idxs.append(ij.reshape(1, T))

    v = jnp.concatenate(vals, axis=0)
    i = jnp.concatenate(idxs, axis=0)
    wsum = jnp.sum(v, axis=0, keepdims=True)
    wout_ref[...] = v / wsum * ROUTE_SCALE
    iout_ref[...] = i


@jax.jit
def kernel(x, weight, bias):
    b2 = bias.reshape(E, 1)
    grid = (N // T,)
    wT, iT = pl.pallas_call(
        _gate_kernel,
        grid=grid,
        in_specs=[
            pl.BlockSpec((T, DIM), lambda i: (i, 0)),
            pl.BlockSpec((E, DIM), lambda i: (0, 0)),
            pl.BlockSpec((E, 1), lambda i: (0, 0)),
        ],
        out_specs=[
            pl.BlockSpec((TOPK, T), lambda i: (0, i)),
            pl.BlockSpec((TOPK, T), lambda i: (0, i)),
        ],
        out_shape=[
            jax.ShapeDtypeStruct((TOPK, N), jnp.float32),
            jax.ShapeDtypeStruct((TOPK, N), jnp.int32),
        ],
    )(x, weight, b2)
    return wT.T, iT.T


# speedup vs baseline: 11.9240x; 1.3708x over previous
import jax
import jax.numpy as jnp
from jax import lax
from jax.experimental import pallas as pl

N = 32768
DIM = 768
E = 64
TOPK = 8
G = 8
EPG = E // G
TOPK_G = 4
ROUTE_SCALE = 2.5

T = 512

NEG = float("-inf")


def _top2_sum(s3):
    half = s3.shape[0] // 2
    h = jnp.maximum(s3[:half], s3[half:])
    l = jnp.minimum(s3[:half], s3[half:])
    while h.shape[0] > 1:
        half = h.shape[0] // 2
        h1, h2 = h[:half], h[half:]
        l1, l2 = l[:half], l[half:]
        hi = jnp.maximum(h1, h2)
        lo = jnp.minimum(h1, h2)
        lw = jnp.where(h1 >= h2, l1, l2)
        h, l = hi, jnp.maximum(lo, lw)
    return (h + l)[0]


def _gate_kernel(x_ref, w_ref, b_ref, wout_ref, iout_ref):
    x = x_ref[...]
    w = w_ref[...]
    b = b_ref[...]

    logits = lax.dot_general(w, x, (((1,), (1,)), ((), ())),
                             preferred_element_type=jnp.float32)
    sig = jax.nn.sigmoid(logits)
    sel = sig + b

    sig3 = sig.reshape(EPG, G, T)
    sel3 = sel.reshape(EPG, G, T)

    gs = _top2_sum(sel3)

    rank = jnp.zeros((G, T), dtype=jnp.int32)
    for h in range(G):
        gh = gs[h:h + 1, :]
        rank = rank + (gh > gs).astype(jnp.int32)
        if h + 1 < G:
            ties = (gs[h + 1:, :] == gh).astype(jnp.int32)
            rank = rank + jnp.concatenate(
                [jnp.zeros((h + 1, T), jnp.int32), ties], axis=0)
    keep = rank < TOPK_G

    sm = jnp.where(keep[None, :, :], sel3, NEG)

    ir = lax.broadcasted_iota(jnp.int32, (EPG, G, T), 0)
    ig = lax.broadcasted_iota(jnp.int32, (EPG, G, T), 1)
    eid = ig * EPG + ir

    vals = []
    idxs = []
    for _ in range(TOPK):
        m = jnp.max(jnp.max(sm, axis=0), axis=0, keepdims=True)
        mb = m[None, :, :]
        cand = jnp.where(sm == mb, eid, E)
        ij = jnp.min(jnp.min(cand, axis=0), axis=0, keepdims=True)
        hit = eid == ij[None, :, :]
        vo = jnp.max(jnp.max(jnp.where(hit, sig3, NEG), axis=0),
                     axis=0, keepdims=True)
        sm = jnp.where(hit, NEG, sm)
        vals.append(vo)
        idxs.append(ij)

    v = jnp.concatenate(vals, axis=0)
    i = jnp.concatenate(idxs, axis=0)
    wsum = jnp.sum(v, axis=0, keepdims=True)
    wout_ref[...] = v / wsum * ROUTE_SCALE
    iout_ref[...] = i


@jax.jit
def kernel(x, weight, bias):
    wp = weight.reshape(G, EPG, DIM).transpose(1, 0, 2).reshape(E, DIM)
    bp = bias.reshape(G, EPG).T.reshape(E, 1)
    grid = (N // T,)
    wT, iT = pl.pallas_call(
        _gate_kernel,
        grid=grid,
        in_specs=[
            pl.BlockSpec((T, DIM), lambda i: (i, 0)),
            pl.BlockSpec((E, DIM), lambda i: (0, 0)),
            pl.BlockSpec((E, 1), lambda i: (0, 0)),
        ],
        out_specs=[
            pl.BlockSpec((TOPK, T), lambda i: (0, i)),
            pl.BlockSpec((TOPK, T), lambda i: (0, i)),
        ],
        out_shape=[
            jax.ShapeDtypeStruct((TOPK, N), jnp.float32),
            jax.ShapeDtypeStruct((TOPK, N), jnp.int32),
        ],
    )(x, wp, bp)
    return wT.T, iT.T
